# Initial kernel scaffold; baseline (speedup 1.0000x reference)
import jax
import jax.numpy as jnp
from jax.experimental import pallas as pl


def kernel(rects, img):
    raise NotImplementedError("write your pallas kernel here")


# speedup vs baseline: 3.0200x; 3.0200x over previous
import jax
import jax.numpy as jnp
from jax import lax
from jax.experimental import pallas as pl

MAX_OUT = 100
IOU_THR = 0.7
OUT_SIZE = 24
WIN = 128
NEG = float("-inf")


def _nms_crop_body(rects_ref, img_ref, crops_ref, bb_ref):
    R8, CN = rects_ref.shape[2], rects_ref.shape[3]
    H, W = img_ref.shape[2], img_ref.shape[3]

    x1a = rects_ref[0, 0]
    y1a = rects_ref[0, 1]
    x2a = rects_ref[0, 2]
    y2a = rects_ref[0, 3]
    scores0 = rects_ref[0, 4]
    areas = (x2a - x1a) * (y2a - y1a)
    flat = (lax.broadcasted_iota(jnp.int32, (R8, CN), 0) * CN
            + lax.broadcasted_iota(jnp.int32, (R8, CN), 1))
    bigint = jnp.int32(2 ** 30)

    def step(i, scores_w):
        m = jnp.max(scores_w)
        valid = m > NEG
        idxsel = jnp.min(jnp.where(scores_w == m, flat, bigint))
        mask = flat == idxsel
        bx1 = jnp.max(jnp.where(mask, x1a, NEG))
        by1 = jnp.max(jnp.where(mask, y1a, NEG))
        bx2 = jnp.max(jnp.where(mask, x2a, NEG))
        by2 = jnp.max(jnp.where(mask, y2a, NEG))

        ix1 = jnp.maximum(bx1, x1a)
        iy1 = jnp.maximum(by1, y1a)
        ix2 = jnp.minimum(bx2, x2a)
        iy2 = jnp.minimum(by2, y2a)
        inter = jnp.maximum(ix2 - ix1, 0.0) * jnp.maximum(iy2 - iy1, 0.0)
        area_b = (bx2 - bx1) * (by2 - by1)
        iou = inter / (area_b + areas - inter)
        supp = (iou > IOU_THR) & valid
        scores_new = jnp.where(supp, NEG, scores_w)

        vx1 = jnp.where(valid, bx1, 0.0)
        vy1 = jnp.where(valid, by1, 0.0)
        vx2 = jnp.where(valid, bx2, 0.0)
        vy2 = jnp.where(valid, by2, 0.0)
        vm = jnp.where(valid, m, 0.0)
        lane8 = lax.broadcasted_iota(jnp.int32, (1, 8), 1)
        row = (jnp.where(lane8 == 0, vx1, 0.0)
               + jnp.where(lane8 == 1, vy1, 0.0)
               + jnp.where(lane8 == 2, vx2, 0.0)
               + jnp.where(lane8 == 3, vy2, 0.0)
               + jnp.where(lane8 == 4, vm, 0.0))
        bb_ref[0, pl.ds(i, 1), :] = row

        x1q = vx1.astype(jnp.int32)
        y1q = vy1.astype(jnp.int32)
        x2q = vx2.astype(jnp.int32)
        y2q = vy2.astype(jnp.int32)
        h = (y2q - y1q).astype(jnp.float32)
        w = (x2q - x1q).astype(jnp.float32)
        y0 = (y1q - 1).astype(jnp.float32)
        x0 = (x1q - 1).astype(jnp.float32)

        iic = lax.broadcasted_iota(jnp.int32, (OUT_SIZE, 1), 0).astype(jnp.float32)
        sy = y0 + iic * h / OUT_SIZE
        yf = jnp.floor(sy)
        wy = sy - yf
        y0i = jnp.clip(yf, 0, H - 1).astype(jnp.int32)
        y1i = jnp.clip(yf + 1.0, 0, H - 1).astype(jnp.int32)
        ys = jnp.clip(y1q - 1, 0, H - WIN)
        ys = pl.multiple_of((ys // 8) * 8, 8)
        y0r = y0i - ys
        y1r = y1i - ys
        jj = lax.broadcasted_iota(jnp.int32, (OUT_SIZE, WIN), 1)
        ry = (jnp.where(jj == y0r, 1.0 - wy, 0.0)
              + jnp.where(jj == y1r, wy, 0.0))

        sx = x0 + iic * w / OUT_SIZE
        xf = jnp.floor(sx)
        wx = sx - xf
        x0i = jnp.clip(xf, 0, W - 1).astype(jnp.int32)
        x1i = jnp.clip(xf + 1.0, 0, W - 1).astype(jnp.int32)
        kk = lax.broadcasted_iota(jnp.int32, (OUT_SIZE, W), 1)
        rx = (jnp.where(kk == x0i, 1.0 - wx, 0.0)
              + jnp.where(kk == x1i, wx, 0.0))

        for c in range(3):
            win = img_ref[0, c, pl.ds(ys, WIN), :]
            b1 = lax.dot_general(
                win, rx, (((1,), (1,)), ((), ())),
                precision=lax.Precision.HIGHEST,
                preferred_element_type=jnp.float32)
            outc = lax.dot_general(
                ry, b1, (((1,), (0,)), ((), ())),
                precision=lax.Precision.HIGHEST,
                preferred_element_type=jnp.float32)
            crops_ref[0, c, pl.ds(i, 1), :, :] = outc.reshape(1, OUT_SIZE, OUT_SIZE)
        return scores_new

    lax.fori_loop(0, MAX_OUT, step, scores0)


def kernel(rects, img):
    B, N, _ = rects.shape
    _, H, W, C = img.shape
    R8 = 8
    CN = N // R8
    rects_t = rects.transpose(0, 2, 1).reshape(B, 5, R8, CN)
    img_t = img.transpose(0, 3, 1, 2)
    crops_t, bb8 = pl.pallas_call(
        _nms_crop_body,
        grid=(B,),
        in_specs=[
            pl.BlockSpec((1, 5, R8, CN), lambda b: (b, 0, 0, 0)),
            pl.BlockSpec((1, C, H, W), lambda b: (b, 0, 0, 0)),
        ],
        out_specs=[
            pl.BlockSpec((1, C, MAX_OUT, OUT_SIZE, OUT_SIZE),
                         lambda b: (b, 0, 0, 0, 0)),
            pl.BlockSpec((1, MAX_OUT, 8), lambda b: (b, 0, 0)),
        ],
        out_shape=[
            jax.ShapeDtypeStruct((B, C, MAX_OUT, OUT_SIZE, OUT_SIZE),
                                 jnp.float32),
            jax.ShapeDtypeStruct((B, MAX_OUT, 8), jnp.float32),
        ],
    )(rects_t, img_t)
    crops = crops_t.transpose(0, 2, 3, 4, 1)
    bb = bb8[..., :5]
    return crops, bb


# speedup vs baseline: 10.9859x; 3.6377x over previous
import jax
import jax.numpy as jnp
from jax import lax
from jax.experimental import pallas as pl
from jax.experimental.pallas import tpu as pltpu

MAX_OUT = 100
IOU_THR = 0.7
OUT_SIZE = 24
NEG = float("-inf")
G = 10
NCHUNK = MAX_OUT // G


def _nms_step(b, i, scores_w, x1a, y1a, x2a, y2a, areas, flat, bb_ref, sm_ref):
    bigint = jnp.int32(2 ** 30)
    m = jnp.max(scores_w)
    valid = m > NEG
    idxsel = jnp.min(jnp.where(scores_w == m, flat, bigint))
    mask = flat == idxsel
    bx1 = jnp.max(jnp.where(mask, x1a, NEG))
    by1 = jnp.max(jnp.where(mask, y1a, NEG))
    bx2 = jnp.max(jnp.where(mask, x2a, NEG))
    by2 = jnp.max(jnp.where(mask, y2a, NEG))

    ix1 = jnp.maximum(bx1, x1a)
    iy1 = jnp.maximum(by1, y1a)
    ix2 = jnp.minimum(bx2, x2a)
    iy2 = jnp.minimum(by2, y2a)
    inter = jnp.maximum(ix2 - ix1, 0.0) * jnp.maximum(iy2 - iy1, 0.0)
    area_b = (bx2 - bx1) * (by2 - by1)
    iou = inter / (area_b + areas - inter)
    supp = (iou > IOU_THR) & valid
    scores_new = jnp.where(supp, NEG, scores_w)

    vx1 = jnp.where(valid, bx1, 0.0)
    vy1 = jnp.where(valid, by1, 0.0)
    vx2 = jnp.where(valid, bx2, 0.0)
    vy2 = jnp.where(valid, by2, 0.0)
    vm = jnp.where(valid, m, 0.0)
    lane8 = lax.broadcasted_iota(jnp.int32, (1, 8), 1)
    row = (jnp.where(lane8 == 0, vx1, 0.0)
           + jnp.where(lane8 == 1, vy1, 0.0)
           + jnp.where(lane8 == 2, vx2, 0.0)
           + jnp.where(lane8 == 3, vy2, 0.0)
           + jnp.where(lane8 == 4, vm, 0.0))
    bb_ref[b, pl.ds(i, 1), :] = row
    sm_ref[b, i, 0] = vx1
    sm_ref[b, i, 1] = vy1
    sm_ref[b, i, 2] = vx2
    sm_ref[b, i, 3] = vy2
    return scores_new


def _body(rects_ref, img_ref, crops_ref, bb_ref, sm_ref):
    B = rects_ref.shape[0]
    R8, CN = rects_ref.shape[2], rects_ref.shape[3]
    H, W = img_ref.shape[2], img_ref.shape[3]

    flat = (lax.broadcasted_iota(jnp.int32, (R8, CN), 0) * CN
            + lax.broadcasted_iota(jnp.int32, (R8, CN), 1))

    def step(i, carry):
        new = []
        for b in range(B):
            x1a = rects_ref[b, 0]
            y1a = rects_ref[b, 1]
            x2a = rects_ref[b, 2]
            y2a = rects_ref[b, 3]
            areas = (x2a - x1a) * (y2a - y1a)
            new.append(_nms_step(b, i, carry[b], x1a, y1a, x2a, y2a,
                                 areas, flat, bb_ref, sm_ref))
        return tuple(new)

    lax.fori_loop(0, MAX_OUT, step,
                  tuple(rects_ref[b, 4] for b in range(B)))

    iic = lax.broadcasted_iota(jnp.int32, (OUT_SIZE, 1), 0).astype(jnp.float32)
    jj = lax.broadcasted_iota(jnp.int32, (G * OUT_SIZE, H), 1)
    kk = lax.broadcasted_iota(jnp.int32, (G * OUT_SIZE, W), 1)

    def chunk(k, _):
        for b in range(B):
            y0i_l, y1i_l, wy_l, x0i_l, x1i_l, wx_l = [], [], [], [], [], []
            for g in range(G):
                idx = k * G + g
                x1q = sm_ref[b, idx, 0].astype(jnp.int32)
                y1q = sm_ref[b, idx, 1].astype(jnp.int32)
                x2q = sm_ref[b, idx, 2].astype(jnp.int32)
                y2q = sm_ref[b, idx, 3].astype(jnp.int32)
                h = (y2q - y1q).astype(jnp.float32)
                w = (x2q - x1q).astype(jnp.float32)
                y0 = (y1q - 1).astype(jnp.float32)
                x0 = (x1q - 1).astype(jnp.float32)
                sy = y0 + iic * h / OUT_SIZE
                yf = jnp.floor(sy)
                wy_l.append(sy - yf)
                y0i_l.append(jnp.clip(yf, 0, H - 1).astype(jnp.int32))
                y1i_l.append(jnp.clip(yf + 1.0, 0, H - 1).astype(jnp.int32))
                sx = x0 + iic * w / OUT_SIZE
                xf = jnp.floor(sx)
                wx_l.append(sx - xf)
                x0i_l.append(jnp.clip(xf, 0, W - 1).astype(jnp.int32))
                x1i_l.append(jnp.clip(xf + 1.0, 0, W - 1).astype(jnp.int32))
            y0col = jnp.concatenate(y0i_l, axis=0)
            y1col = jnp.concatenate(y1i_l, axis=0)
            wycol = jnp.concatenate(wy_l, axis=0)
            x0col = jnp.concatenate(x0i_l, axis=0)
            x1col = jnp.concatenate(x1i_l, axis=0)
            wxcol = jnp.concatenate(wx_l, axis=0)
            ry = (jnp.where(jj == y0col, 1.0 - wycol, 0.0)
                  + jnp.where(jj == y1col, wycol, 0.0)).astype(jnp.bfloat16)
            rx = (jnp.where(kk == x0col, 1.0 - wxcol, 0.0)
                  + jnp.where(kk == x1col, wxcol, 0.0)).astype(jnp.bfloat16)
            for c in range(3):
                im = img_ref[b, c]
                p = lax.dot_general(
                    im, rx, (((1,), (1,)), ((), ())),
                    preferred_element_type=jnp.float32).astype(jnp.bfloat16)
                big = lax.dot_general(
                    ry, p, (((1,), (0,)), ((), ())),
                    preferred_element_type=jnp.float32)
                for g in range(G):
                    outc = big[g * OUT_SIZE:(g + 1) * OUT_SIZE,
                               g * OUT_SIZE:(g + 1) * OUT_SIZE]
                    crops_ref[b, c, pl.ds(k * G + g, 1), :, :] = (
                        outc.reshape(1, OUT_SIZE, OUT_SIZE))
        return 0

    lax.fori_loop(0, NCHUNK, chunk, 0)


def kernel(rects, img):
    B, N, _ = rects.shape
    _, H, W, C = img.shape
    R8 = 8
    CN = N // R8
    rects_t = rects.transpose(0, 2, 1).reshape(B, 5, R8, CN)
    img_t = img.transpose(0, 3, 1, 2).astype(jnp.bfloat16)
    crops_t, bb8 = pl.pallas_call(
        _body,
        grid=(1,),
        in_specs=[
            pl.BlockSpec((B, 5, R8, CN), lambda b: (0, 0, 0, 0)),
            pl.BlockSpec((B, C, H, W), lambda b: (0, 0, 0, 0)),
        ],
        out_specs=[
            pl.BlockSpec((B, C, MAX_OUT, OUT_SIZE, OUT_SIZE),
                         lambda b: (0, 0, 0, 0, 0)),
            pl.BlockSpec((B, MAX_OUT, 8), lambda b: (0, 0, 0)),
        ],
        out_shape=[
            jax.ShapeDtypeStruct((B, C, MAX_OUT, OUT_SIZE, OUT_SIZE),
                                 jnp.float32),
            jax.ShapeDtypeStruct((B, MAX_OUT, 8), jnp.float32),
        ],
        scratch_shapes=[pltpu.SMEM((B, MAX_OUT, 8), jnp.float32)],
    )(rects_t, img_t)
    crops = crops_t.transpose(0, 2, 3, 4, 1)
    bb = bb8[..., :5]
    return crops, bb


# speedup vs baseline: 15.3013x; 1.3928x over previous
import jax
import jax.numpy as jnp
from jax import lax
from jax.experimental import pallas as pl
from jax.experimental.pallas import tpu as pltpu

MAX_OUT = 100
IOU_THR = 0.7
OUT_SIZE = 24
NEG = float("-inf")
G = 10
NCHUNK = MAX_OUT // G


def _body(rects_ref, coordt_ref, img_ref, crops_ref, bb_ref, sm_ref, area_ref):
    B = rects_ref.shape[0]
    R8, CN = rects_ref.shape[2], rects_ref.shape[3]
    H, W = img_ref.shape[2], img_ref.shape[3]

    flat = (lax.broadcasted_iota(jnp.int32, (R8, CN), 0) * CN
            + lax.broadcasted_iota(jnp.int32, (R8, CN), 1))
    bigint = jnp.int32(2 ** 30)
    subl8 = lax.broadcasted_iota(jnp.int32, (8, 8), 0)
    lane8g = lax.broadcasted_iota(jnp.int32, (8, 8), 1)
    lane8 = lax.broadcasted_iota(jnp.int32, (1, 8), 1)

    for b in range(B):
        area_ref[b] = ((rects_ref[b, 2] - rects_ref[b, 0])
                       * (rects_ref[b, 3] - rects_ref[b, 1]))

    def step(i, carry):
        ms = [jnp.max(carry[b]) for b in range(B)]
        idxs = [jnp.min(jnp.where(carry[b] == ms[b], flat, bigint))
                for b in range(B)]
        outs = []
        for b in range(B):
            m, idxsel = ms[b], idxs[b]
            valid = m > NEG
            cc = idxsel % CN
            rr = idxsel // CN
            ca = pl.multiple_of((cc // 8) * 8, 8)
            ss = cc - ca
            pick = (subl8 == ss) & (lane8g == rr)
            bx1 = jnp.sum(jnp.where(pick, coordt_ref[b, 0, pl.ds(ca, 8), :], 0.0))
            by1 = jnp.sum(jnp.where(pick, coordt_ref[b, 1, pl.ds(ca, 8), :], 0.0))
            bx2 = jnp.sum(jnp.where(pick, coordt_ref[b, 2, pl.ds(ca, 8), :], 0.0))
            by2 = jnp.sum(jnp.where(pick, coordt_ref[b, 3, pl.ds(ca, 8), :], 0.0))

            x1a = rects_ref[b, 0]
            y1a = rects_ref[b, 1]
            x2a = rects_ref[b, 2]
            y2a = rects_ref[b, 3]
            ix1 = jnp.maximum(bx1, x1a)
            iy1 = jnp.maximum(by1, y1a)
            ix2 = jnp.minimum(bx2, x2a)
            iy2 = jnp.minimum(by2, y2a)
            inter = (jnp.maximum(ix2 - ix1, 0.0)
                     * jnp.maximum(iy2 - iy1, 0.0))
            area_b = (bx2 - bx1) * (by2 - by1)
            iou = inter / (area_b + area_ref[b] - inter)
            supp = (iou > IOU_THR) & valid
            outs.append(jnp.where(supp, NEG, carry[b]))

            vx1 = jnp.where(valid, bx1, 0.0)
            vy1 = jnp.where(valid, by1, 0.0)
            vx2 = jnp.where(valid, bx2, 0.0)
            vy2 = jnp.where(valid, by2, 0.0)
            vm = jnp.where(valid, m, 0.0)
            row = (jnp.where(lane8 == 0, vx1, 0.0)
                   + jnp.where(lane8 == 1, vy1, 0.0)
                   + jnp.where(lane8 == 2, vx2, 0.0)
                   + jnp.where(lane8 == 3, vy2, 0.0)
                   + jnp.where(lane8 == 4, vm, 0.0))
            bb_ref[b, pl.ds(i, 1), :] = row
            sm_ref[b, i, 0] = vx1
            sm_ref[b, i, 1] = vy1
            sm_ref[b, i, 2] = vx2
            sm_ref[b, i, 3] = vy2
        return tuple(outs)

    lax.fori_loop(0, MAX_OUT, step,
                  tuple(rects_ref[b, 4] for b in range(B)))

    iic = lax.broadcasted_iota(jnp.int32, (OUT_SIZE, 1), 0).astype(jnp.float32)
    jj = lax.broadcasted_iota(jnp.int32, (G * OUT_SIZE, H), 1)
    kk = lax.broadcasted_iota(jnp.int32, (G * OUT_SIZE, W), 1)

    def chunk(k, _):
        for b in range(B):
            y0i_l, y1i_l, wy_l, x0i_l, x1i_l, wx_l = [], [], [], [], [], []
            for g in range(G):
                idx = k * G + g
                x1q = sm_ref[b, idx, 0].astype(jnp.int32)
                y1q = sm_ref[b, idx, 1].astype(jnp.int32)
                x2q = sm_ref[b, idx, 2].astype(jnp.int32)
                y2q = sm_ref[b, idx, 3].astype(jnp.int32)
                h = (y2q - y1q).astype(jnp.float32)
                w = (x2q - x1q).astype(jnp.float32)
                y0 = (y1q - 1).astype(jnp.float32)
                x0 = (x1q - 1).astype(jnp.float32)
                sy = y0 + iic * h / OUT_SIZE
                yf = jnp.floor(sy)
                wy_l.append(sy - yf)
                y0i_l.append(jnp.clip(yf, 0, H - 1).astype(jnp.int32))
                y1i_l.append(jnp.clip(yf + 1.0, 0, H - 1).astype(jnp.int32))
                sx = x0 + iic * w / OUT_SIZE
                xf = jnp.floor(sx)
                wx_l.append(sx - xf)
                x0i_l.append(jnp.clip(xf, 0, W - 1).astype(jnp.int32))
                x1i_l.append(jnp.clip(xf + 1.0, 0, W - 1).astype(jnp.int32))
            y0col = jnp.concatenate(y0i_l, axis=0)
            y1col = jnp.concatenate(y1i_l, axis=0)
            wycol = jnp.concatenate(wy_l, axis=0)
            x0col = jnp.concatenate(x0i_l, axis=0)
            x1col = jnp.concatenate(x1i_l, axis=0)
            wxcol = jnp.concatenate(wx_l, axis=0)
            ry = (jnp.where(jj == y0col, 1.0 - wycol, 0.0)
                  + jnp.where(jj == y1col, wycol, 0.0)).astype(jnp.bfloat16)
            rx = (jnp.where(kk == x0col, 1.0 - wxcol, 0.0)
                  + jnp.where(kk == x1col, wxcol, 0.0)).astype(jnp.bfloat16)
            for c in range(3):
                im = img_ref[b, c]
                p = lax.dot_general(
                    im, rx, (((1,), (1,)), ((), ())),
                    preferred_element_type=jnp.float32).astype(jnp.bfloat16)
                big = lax.dot_general(
                    ry, p, (((1,), (0,)), ((), ())),
                    preferred_element_type=jnp.float32)
                for g in range(G):
                    outc = big[g * OUT_SIZE:(g + 1) * OUT_SIZE,
                               g * OUT_SIZE:(g + 1) * OUT_SIZE]
                    crops_ref[b, c, pl.ds(k * G + g, 1), :, :] = (
                        outc.reshape(1, OUT_SIZE, OUT_SIZE))
        return 0

    lax.fori_loop(0, NCHUNK, chunk, 0)


def kernel(rects, img):
    B, N, _ = rects.shape
    _, H, W, C = img.shape
    R8 = 8
    CN = N // R8
    rects_t = rects.transpose(0, 2, 1).reshape(B, 5, R8, CN)
    cpad = (-CN) % 8
    coordt = rects[:, :, :4].reshape(B, R8, CN, 4).transpose(0, 3, 2, 1)
    coordt = jnp.pad(coordt, ((0, 0), (0, 0), (0, cpad), (0, 0)))
    img_t = img.transpose(0, 3, 1, 2).astype(jnp.bfloat16)
    crops_t, bb8 = pl.pallas_call(
        _body,
        grid=(1,),
        in_specs=[
            pl.BlockSpec((B, 5, R8, CN), lambda b: (0, 0, 0, 0)),
            pl.BlockSpec((B, 4, CN + cpad, R8), lambda b: (0, 0, 0, 0)),
            pl.BlockSpec((B, C, H, W), lambda b: (0, 0, 0, 0)),
        ],
        out_specs=[
            pl.BlockSpec((B, C, MAX_OUT, OUT_SIZE, OUT_SIZE),
                         lambda b: (0, 0, 0, 0, 0)),
            pl.BlockSpec((B, MAX_OUT, 8), lambda b: (0, 0, 0)),
        ],
        out_shape=[
            jax.ShapeDtypeStruct((B, C, MAX_OUT, OUT_SIZE, OUT_SIZE),
                                 jnp.float32),
            jax.ShapeDtypeStruct((B, MAX_OUT, 8), jnp.float32),
        ],
        scratch_shapes=[pltpu.SMEM((B, MAX_OUT, 8), jnp.float32),
                        pltpu.VMEM((B, R8, CN), jnp.float32)],
    )(rects_t, coordt, img_t)
    crops = crops_t.transpose(0, 2, 3, 4, 1)
    bb = bb8[..., :5]
    return crops, bb


# speedup vs baseline: 15.3021x; 1.0001x over previous
import jax
import jax.numpy as jnp
from jax import lax
from jax.experimental import pallas as pl
from jax.experimental.pallas import tpu as pltpu

MAX_OUT = 100
IOU_THR = 0.7
OUT_SIZE = 24
NEG = float("-inf")
G = 10
NCHUNK = MAX_OUT // G


def _body(rects_ref, coordt_ref, img_ref, crops_ref, bb_ref, sm_ref, area_ref):
    B = rects_ref.shape[0]
    R8, CN = rects_ref.shape[2], rects_ref.shape[3]
    H, W = img_ref.shape[2], img_ref.shape[3]

    flat = (lax.broadcasted_iota(jnp.int32, (R8, CN), 0) * CN
            + lax.broadcasted_iota(jnp.int32, (R8, CN), 1))
    bigint = jnp.int32(2 ** 30)
    subl8 = lax.broadcasted_iota(jnp.int32, (8, 8), 0)
    lane8g = lax.broadcasted_iota(jnp.int32, (8, 8), 1)
    lane8 = lax.broadcasted_iota(jnp.int32, (1, 8), 1)

    for b in range(B):
        area_ref[b] = ((rects_ref[b, 2] - rects_ref[b, 0])
                       * (rects_ref[b, 3] - rects_ref[b, 1]))

    def step(i, carry):
        ms = [jnp.max(carry[b]) for b in range(B)]
        idxs = [jnp.min(jnp.where(carry[b] == ms[b], flat, bigint))
                for b in range(B)]
        outs = []
        for b in range(B):
            m, idxsel = ms[b], idxs[b]
            valid = m > NEG
            cc = idxsel % CN
            rr = idxsel // CN
            ca = pl.multiple_of((cc // 8) * 8, 8)
            ss = cc - ca
            pick = (subl8 == ss) & (lane8g == rr)
            bx1 = jnp.sum(jnp.where(pick, coordt_ref[b, 0, pl.ds(ca, 8), :], 0.0))
            by1 = jnp.sum(jnp.where(pick, coordt_ref[b, 1, pl.ds(ca, 8), :], 0.0))
            bx2 = jnp.sum(jnp.where(pick, coordt_ref[b, 2, pl.ds(ca, 8), :], 0.0))
            by2 = jnp.sum(jnp.where(pick, coordt_ref[b, 3, pl.ds(ca, 8), :], 0.0))

            x1a = rects_ref[b, 0]
            y1a = rects_ref[b, 1]
            x2a = rects_ref[b, 2]
            y2a = rects_ref[b, 3]
            ix1 = jnp.maximum(bx1, x1a)
            iy1 = jnp.maximum(by1, y1a)
            ix2 = jnp.minimum(bx2, x2a)
            iy2 = jnp.minimum(by2, y2a)
            inter = (jnp.maximum(ix2 - ix1, 0.0)
                     * jnp.maximum(iy2 - iy1, 0.0))
            area_b = (bx2 - bx1) * (by2 - by1)
            iou = inter / (area_b + area_ref[b] - inter)
            supp = (iou > IOU_THR) & valid
            outs.append(jnp.where(supp, NEG, carry[b]))

            vx1 = jnp.where(valid, bx1, 0.0)
            vy1 = jnp.where(valid, by1, 0.0)
            vx2 = jnp.where(valid, bx2, 0.0)
            vy2 = jnp.where(valid, by2, 0.0)
            vm = jnp.where(valid, m, 0.0)
            row = (jnp.where(lane8 == 0, vx1, 0.0)
                   + jnp.where(lane8 == 1, vy1, 0.0)
                   + jnp.where(lane8 == 2, vx2, 0.0)
                   + jnp.where(lane8 == 3, vy2, 0.0)
                   + jnp.where(lane8 == 4, vm, 0.0))
            bb_ref[b, pl.ds(i, 1), :] = row
            sm_ref[b, i, 0] = vx1
            sm_ref[b, i, 1] = vy1
            sm_ref[b, i, 2] = vx2
            sm_ref[b, i, 3] = vy2
        return tuple(outs)

    lax.fori_loop(0, MAX_OUT, step,
                  tuple(rects_ref[b, 4] for b in range(B)))

    iic = lax.broadcasted_iota(jnp.int32, (OUT_SIZE, 1), 0).astype(jnp.float32)
    jjf = lax.broadcasted_iota(jnp.int32, (G * OUT_SIZE, H), 1).astype(jnp.float32)
    kkf = lax.broadcasted_iota(jnp.int32, (G * OUT_SIZE, W), 1).astype(jnp.float32)

    def chunk(k, _):
        for b in range(B):
            sy_l, sx_l = [], []
            for g in range(G):
                idx = k * G + g
                x1q = sm_ref[b, idx, 0].astype(jnp.int32)
                y1q = sm_ref[b, idx, 1].astype(jnp.int32)
                x2q = sm_ref[b, idx, 2].astype(jnp.int32)
                y2q = sm_ref[b, idx, 3].astype(jnp.int32)
                h = (y2q - y1q).astype(jnp.float32)
                w = (x2q - x1q).astype(jnp.float32)
                y0 = (y1q - 1).astype(jnp.float32)
                x0 = (x1q - 1).astype(jnp.float32)
                sy_l.append(jnp.maximum(y0 + iic * h / OUT_SIZE, 0.0))
                sx_l.append(jnp.maximum(x0 + iic * w / OUT_SIZE, 0.0))
            sycol = jnp.concatenate(sy_l, axis=0)
            sxcol = jnp.concatenate(sx_l, axis=0)
            ry = jnp.maximum(1.0 - jnp.abs(jjf - sycol), 0.0).astype(jnp.bfloat16)
            rx = jnp.maximum(1.0 - jnp.abs(kkf - sxcol), 0.0).astype(jnp.bfloat16)
            for c in range(3):
                im = img_ref[b, c]
                p = lax.dot_general(
                    im, rx, (((1,), (1,)), ((), ())),
                    preferred_element_type=jnp.float32).astype(jnp.bfloat16)
                big = lax.dot_general(
                    ry, p, (((1,), (0,)), ((), ())),
                    preferred_element_type=jnp.float32)
                for g in range(G):
                    outc = big[g * OUT_SIZE:(g + 1) * OUT_SIZE,
                               g * OUT_SIZE:(g + 1) * OUT_SIZE]
                    crops_ref[b, c, pl.ds(k * G + g, 1), :, :] = (
                        outc.reshape(1, OUT_SIZE, OUT_SIZE))
        return 0

    lax.fori_loop(0, NCHUNK, chunk, 0)


def kernel(rects, img):
    B, N, _ = rects.shape
    _, H, W, C = img.shape
    R8 = 8
    CN = N // R8
    rects_t = rects.transpose(0, 2, 1).reshape(B, 5, R8, CN)
    cpad = (-CN) % 8
    coordt = rects[:, :, :4].reshape(B, R8, CN, 4).transpose(0, 3, 2, 1)
    coordt = jnp.pad(coordt, ((0, 0), (0, 0), (0, cpad), (0, 0)))
    img_t = img.transpose(0, 3, 1, 2).astype(jnp.bfloat16)
    crops_t, bb8 = pl.pallas_call(
        _body,
        grid=(1,),
        in_specs=[
            pl.BlockSpec((B, 5, R8, CN), lambda b: (0, 0, 0, 0)),
            pl.BlockSpec((B, 4, CN + cpad, R8), lambda b: (0, 0, 0, 0)),
            pl.BlockSpec((B, C, H, W), lambda b: (0, 0, 0, 0)),
        ],
        out_specs=[
            pl.BlockSpec((B, C, MAX_OUT, OUT_SIZE, OUT_SIZE),
                         lambda b: (0, 0, 0, 0, 0)),
            pl.BlockSpec((B, MAX_OUT, 8), lambda b: (0, 0, 0)),
        ],
        out_shape=[
            jax.ShapeDtypeStruct((B, C, MAX_OUT, OUT_SIZE, OUT_SIZE),
                                 jnp.float32),
            jax.ShapeDtypeStruct((B, MAX_OUT, 8), jnp.float32),
        ],
        scratch_shapes=[pltpu.SMEM((B, MAX_OUT, 8), jnp.float32),
                        pltpu.VMEM((B, R8, CN), jnp.float32)],
    )(rects_t, coordt, img_t)
    crops = crops_t.transpose(0, 2, 3, 4, 1)
    bb = bb8[..., :5]
    return crops, bb
